# baseline (speedup 1.0000x reference)
import functools

import jax
import jax.numpy as jnp
from jax import lax
from jax.experimental import pallas as pl
from jax.experimental.pallas import tpu as pltpu
from jax.experimental.pallas import tpu_sc as plsc

NW = 32
L = 16


@functools.lru_cache(maxsize=None)
def _build_transpose(D, V):
    TB = 256
    VPW = -(-V // (NW * 8)) * 8
    NB = (VPW + TB - 1) // TB

    mesh = plsc.VectorSubcoreMesh(core_axis_name="c", subcore_axis_name="s")

    @functools.partial(
        pl.kernel,
        out_type=jax.ShapeDtypeStruct((V * D,), jnp.float32),
        mesh=mesh,
        scratch_types=[
            pltpu.VMEM((D, TB), jnp.float32),
            pltpu.VMEM((TB * D,), jnp.float32),
        ],
        compiler_params=pltpu.CompilerParams(
            use_tc_tiling_on_sc=False, needs_layout_passes=False),
    )
    def tr(wt_hbm, wlin_hbm, wt_v, out_v):
        cid = lax.axis_index("c")
        sid = lax.axis_index("s")
        wid = sid * 2 + cid
        lanes32 = lax.iota(jnp.int32, L) * D

        def block_body(bb, c2):
            t0 = jnp.minimum(wid * VPW + bb * TB, V - TB)
            pltpu.sync_copy(wt_hbm.at[:, pl.ds(t0, TB)], wt_v)

            def group_body(g, c3):
                base = g * (L * D)
                for d in range(D):
                    val = wt_v[d, pl.ds(g * L, L)]
                    plsc.store_scatter(out_v, [lanes32 + (base + d)], val)
                return c3
            lax.fori_loop(0, TB // L, group_body, 0)

            pltpu.sync_copy(out_v, wlin_hbm.at[pl.ds(t0 * D, TB * D)])
            return c2
        lax.fori_loop(0, NB, block_body, 0)

    return tr


@functools.lru_cache(maxsize=None)
def _build(B, S, D, V):
    SPW = B // NW
    C = 8
    NCHUNK = SPW // C
    CHW = C * S
    GB = 80
    NBLK = CHW // GB

    mesh = plsc.VectorSubcoreMesh(core_axis_name="c", subcore_axis_name="s")

    @functools.partial(
        pl.kernel,
        out_type=jax.ShapeDtypeStruct((B * D,), jnp.float32),
        mesh=mesh,
        scratch_types=[
            pltpu.VMEM((2 * CHW,), jnp.int32),
            pltpu.VMEM((2 * CHW,), jnp.int32),
            pltpu.VMEM((2 * CHW,), jnp.int32),
            pltpu.VMEM((2 * CHW, D), jnp.float32),
            pltpu.VMEM((C * D,), jnp.float32),
            pltpu.SemaphoreType.DMA,
            pltpu.SemaphoreType.DMA,
            pltpu.SemaphoreType.DMA,
            pltpu.SemaphoreType.DMA,
        ],
        compiler_params=pltpu.CompilerParams(
            use_tc_tiling_on_sc=False, needs_layout_passes=False),
    )
    def enc(tok_hbm, mask_hbm, table_hbm, out_hbm,
            tok_v, mask_v, idx_v, rows_v, outb_v,
            sem_r0, sem_r1, sem_t0, sem_t1):
        cid = lax.axis_index("c")
        sid = lax.axis_index("s")
        wid = sid * 2 + cid

        lanes = lax.iota(jnp.int32, L)
        first8 = jnp.minimum(jnp.maximum(8 - lanes, 0), 1)
        last8 = 1 - first8
        zero16 = jnp.zeros((L,), jnp.int32)
        zf16 = jnp.zeros((L,), jnp.float32)
        sem_r = (sem_r0, sem_r1)
        sem_t = (sem_t0, sem_t1)

        def seed_body(kk, c2):
            for u in range(4):
                off = (kk * 4 + u) * L
                idx_v[pl.ds(off, L)] = lanes + off
            return c2
        lax.fori_loop(0, 2 * CHW // (4 * L), seed_body, 0)

        def load_tm(ci, b):
            tbase = (wid * SPW + ci * C) * S
            tb = b * CHW
            pltpu.make_async_copy(
                tok_hbm.at[pl.ds(tbase, CHW)],
                tok_v.at[pl.ds(tb, CHW)], sem_t[b]).start()
            pltpu.make_async_copy(
                mask_hbm.at[pl.ds(tbase, CHW)],
                mask_v.at[pl.ds(tb, CHW)], sem_t[b]).start()

        def wait_tm(b):
            tb = b * CHW
            pltpu.make_async_copy(
                tok_hbm.at[pl.ds(0, CHW)],
                tok_v.at[pl.ds(tb, CHW)], sem_t[b]).wait()
            pltpu.make_async_copy(
                mask_hbm.at[pl.ds(0, CHW)],
                mask_v.at[pl.ds(tb, CHW)], sem_t[b]).wait()

        def compact(b):
            tb = b * CHW

            def emit(off_vec, k, sub):
                t = tok_v[pl.ds(tb + k * L, L)]
                m = mask_v[pl.ds(tb + k * L, L)]
                if sub == 0:
                    ms = m * first8
                elif sub == 1:
                    ms = m * last8
                else:
                    ms = m
                mb = ms != zero16
                pos = off_vec + plsc.cumsum(ms) - 1
                plsc.store_scatter(idx_v.at[pl.ds(tb, CHW)], [pos], t,
                                   mask=mb)
                return off_vec + plsc.all_reduce_population_count(mb)

            off_vec = zero16
            ends = []
            vpp = (2 * S) // L
            for p in range(C // 2):
                base = p * vpp
                for k in range(S // L):
                    off_vec = emit(off_vec, base + k, 2)
                off_vec = emit(off_vec, base + S // L, 0)
                ends.append(jnp.max(off_vec))
                off_vec = emit(off_vec, base + S // L, 1)
                for k in range(S // L + 1, vpp):
                    off_vec = emit(off_vec, base + k, 2)
                ends.append(jnp.max(off_vec))
            return tuple(ends)

        def fire_rows(b, ntot):
            tb = b * CHW

            def fire(bb, c2):
                @pl.when(bb * GB < ntot)
                def _():
                    pltpu.make_async_copy(
                        table_hbm.at[idx_v.at[pl.ds(tb + bb * GB, GB)]],
                        rows_v.at[pl.ds(tb + bb * GB, GB)], sem_r[b]).start()
                return c2
            lax.fori_loop(0, NBLK, fire, 0)

        def drain_rows(b, ntot):
            tb = b * CHW

            def drain(bb, c2):
                @pl.when(bb * GB < ntot)
                def _():
                    pltpu.make_async_copy(
                        table_hbm.at[idx_v.at[pl.ds(tb, GB)]],
                        rows_v.at[pl.ds(tb, GB)], sem_r[b]).wait()
                return c2
            lax.fori_loop(0, NBLK, drain, 0)

        def accum(b, ends, ci):
            rbb = b * CHW
            for s in range(C):
                off_s = jnp.int32(0) if s == 0 else ends[s - 1]
                cnt_s = ends[s] - off_s
                rb0 = rbb + off_s

                def g8(g, a, rb0=rb0):
                    a0, a1 = a
                    rb = rb0 + g * 8
                    for r in range(8):
                        a0 = a0 + rows_v[rb + r, pl.ds(0, L)]
                        a1 = a1 + rows_v[rb + r, pl.ds(L, L)]
                    return (a0, a1)
                acc0, acc1 = lax.fori_loop(0, cnt_s // 8, g8, (zf16, zf16))

                def g1(j, a, rb0=rb0):
                    a0, a1 = a
                    return (a0 + rows_v[rb0 + j, pl.ds(0, L)],
                            a1 + rows_v[rb0 + j, pl.ds(L, L)])
                acc0, acc1 = lax.fori_loop(
                    (cnt_s // 8) * 8, cnt_s, g1, (acc0, acc1))

                cntf = jnp.full((L,), cnt_s.astype(jnp.float32))
                scale = jnp.float32(1.0) / jnp.maximum(cntf, 1.0)
                outb_v[pl.ds(s * D, L)] = acc0 * scale
                outb_v[pl.ds(s * D + L, L)] = acc1 * scale
            pltpu.sync_copy(
                outb_v, out_hbm.at[pl.ds((wid * SPW + ci * C) * D, C * D)])

        load_tm(0, 0)
        wait_tm(0)
        ends0 = compact(0)
        fire_rows(0, ends0[-1])
        load_tm(1, 1)

        def pair_body(pi, carry):
            ci = 2 * pi
            wait_tm(1)
            ends_b1 = compact(1)
            fire_rows(1, ends_b1[-1])
            load_tm(ci + 2, 0)
            drain_rows(0, carry[-1])
            accum(0, carry, ci)

            wait_tm(0)
            ends_b0 = compact(0)
            fire_rows(0, ends_b0[-1])
            load_tm(ci + 3, 1)
            drain_rows(1, ends_b1[-1])
            accum(1, ends_b1, ci + 1)
            return ends_b0
        carry = lax.fori_loop(0, NCHUNK // 2 - 1, pair_body, ends0)

        wait_tm(1)
        ends_last = compact(1)
        fire_rows(1, ends_last[-1])
        drain_rows(0, carry[-1])
        accum(0, carry, NCHUNK - 2)
        drain_rows(1, ends_last[-1])
        accum(1, ends_last, NCHUNK - 1)

    return enc


def kernel(tokens, mask, W):
    B, S = tokens.shape
    V, D = W.shape
    tr = _build_transpose(D, V)
    enc = _build(B, S, D, V)
    wlin = tr(jnp.transpose(W)).reshape(V, D)
    out = enc(tokens.reshape(-1),
              mask.astype(jnp.int32).reshape(-1),
              wlin)
    return out.reshape(B, D)


# speedup vs baseline: 4.2477x; 4.2477x over previous
import functools

import jax
import jax.numpy as jnp
from jax import lax
from jax.experimental import pallas as pl
from jax.experimental.pallas import tpu as pltpu
from jax.experimental.pallas import tpu_sc as plsc

NW = 32
L = 16


@functools.lru_cache(maxsize=None)
def _build(B, S, D, V):
    SPW = B // NW
    C = 8
    NCHUNK = SPW // C
    CHW = C * S
    GB = 80
    NBLK = CHW // GB

    mesh = plsc.VectorSubcoreMesh(core_axis_name="c", subcore_axis_name="s")

    @functools.partial(
        pl.kernel,
        out_type=jax.ShapeDtypeStruct((B * D,), jnp.float32),
        mesh=mesh,
        scratch_types=[
            pltpu.VMEM((2 * CHW,), jnp.int32),
            pltpu.VMEM((2 * CHW,), jnp.int32),
            pltpu.VMEM((2 * CHW,), jnp.int32),
            pltpu.VMEM((2 * CHW, D), jnp.float32),
            pltpu.VMEM((C * D,), jnp.float32),
            pltpu.SemaphoreType.DMA,
            pltpu.SemaphoreType.DMA,
            pltpu.SemaphoreType.DMA,
            pltpu.SemaphoreType.DMA,
        ],
        compiler_params=pltpu.CompilerParams(
            use_tc_tiling_on_sc=False, needs_layout_passes=False),
    )
    def enc(tok_hbm, mask_hbm, table_hbm, out_hbm,
            tok_v, mask_v, idx_v, rows_v, outb_v,
            sem_r0, sem_r1, sem_t0, sem_t1):
        cid = lax.axis_index("c")
        sid = lax.axis_index("s")
        wid = sid * 2 + cid

        lanes = lax.iota(jnp.int32, L)
        first8 = jnp.minimum(jnp.maximum(8 - lanes, 0), 1)
        last8 = 1 - first8
        zero16 = jnp.zeros((L,), jnp.int32)
        zf16 = jnp.zeros((L,), jnp.float32)
        sem_r = (sem_r0, sem_r1)
        sem_t = (sem_t0, sem_t1)

        def seed_body(kk, c2):
            for u in range(4):
                off = (kk * 4 + u) * L
                idx_v[pl.ds(off, L)] = lanes + off
            return c2
        lax.fori_loop(0, 2 * CHW // (4 * L), seed_body, 0)

        def load_tm(ci, b):
            tbase = (wid * SPW + ci * C) * S
            tb = b * CHW
            pltpu.make_async_copy(
                tok_hbm.at[pl.ds(tbase, CHW)],
                tok_v.at[pl.ds(tb, CHW)], sem_t[b]).start()
            pltpu.make_async_copy(
                mask_hbm.at[pl.ds(tbase, CHW)],
                mask_v.at[pl.ds(tb, CHW)], sem_t[b]).start()

        def wait_tm(b):
            tb = b * CHW
            pltpu.make_async_copy(
                tok_hbm.at[pl.ds(0, CHW)],
                tok_v.at[pl.ds(tb, CHW)], sem_t[b]).wait()
            pltpu.make_async_copy(
                mask_hbm.at[pl.ds(0, CHW)],
                mask_v.at[pl.ds(tb, CHW)], sem_t[b]).wait()

        def compact(b):
            tb = b * CHW

            def emit(off_vec, k, sub):
                t = tok_v[pl.ds(tb + k * L, L)]
                m = mask_v[pl.ds(tb + k * L, L)]
                if sub == 0:
                    ms = m * first8
                elif sub == 1:
                    ms = m * last8
                else:
                    ms = m
                mb = ms != zero16
                pos = off_vec + plsc.cumsum(ms) - 1
                plsc.store_scatter(idx_v.at[pl.ds(tb, CHW)], [pos], t,
                                   mask=mb)
                return off_vec + plsc.all_reduce_population_count(mb)

            off_vec = zero16
            ends = []
            vpp = (2 * S) // L
            for p in range(C // 2):
                base = p * vpp
                for k in range(S // L):
                    off_vec = emit(off_vec, base + k, 2)
                off_vec = emit(off_vec, base + S // L, 0)
                ends.append(jnp.max(off_vec))
                off_vec = emit(off_vec, base + S // L, 1)
                for k in range(S // L + 1, vpp):
                    off_vec = emit(off_vec, base + k, 2)
                ends.append(jnp.max(off_vec))
            return tuple(ends)

        def fire_rows(b, ntot):
            tb = b * CHW

            def fire(bb, c2):
                @pl.when(bb * GB < ntot)
                def _():
                    pltpu.make_async_copy(
                        table_hbm.at[idx_v.at[pl.ds(tb + bb * GB, GB)]],
                        rows_v.at[pl.ds(tb + bb * GB, GB)], sem_r[b]).start()
                return c2
            lax.fori_loop(0, NBLK, fire, 0)

        def drain_rows(b, ntot):
            tb = b * CHW

            def drain(bb, c2):
                @pl.when(bb * GB < ntot)
                def _():
                    pltpu.make_async_copy(
                        table_hbm.at[idx_v.at[pl.ds(tb, GB)]],
                        rows_v.at[pl.ds(tb, GB)], sem_r[b]).wait()
                return c2
            lax.fori_loop(0, NBLK, drain, 0)

        def accum(b, ends, ci):
            rbb = b * CHW
            for s in range(C):
                off_s = jnp.int32(0) if s == 0 else ends[s - 1]
                cnt_s = ends[s] - off_s
                rb0 = rbb + off_s

                def g16(g, a, rb0=rb0):
                    a0, a1 = a
                    rb = rb0 + g * 16
                    for r in range(16):
                        a0 = a0 + rows_v[rb + r, pl.ds(0, L)]
                        a1 = a1 + rows_v[rb + r, pl.ds(L, L)]
                    return (a0, a1)
                acc0, acc1 = lax.fori_loop(0, cnt_s // 16, g16, (zf16, zf16))

                def g1(j, a, rb0=rb0):
                    a0, a1 = a
                    return (a0 + rows_v[rb0 + j, pl.ds(0, L)],
                            a1 + rows_v[rb0 + j, pl.ds(L, L)])
                acc0, acc1 = lax.fori_loop(
                    (cnt_s // 16) * 16, cnt_s, g1, (acc0, acc1))

                cntf = jnp.full((L,), cnt_s.astype(jnp.float32))
                scale = jnp.float32(1.0) / jnp.maximum(cntf, 1.0)
                outb_v[pl.ds(s * D, L)] = acc0 * scale
                outb_v[pl.ds(s * D + L, L)] = acc1 * scale
            pltpu.sync_copy(
                outb_v, out_hbm.at[pl.ds((wid * SPW + ci * C) * D, C * D)])

        load_tm(0, 0)
        wait_tm(0)
        ends0 = compact(0)
        fire_rows(0, ends0[-1])
        load_tm(1, 1)

        def pair_body(pi, carry):
            ci = 2 * pi
            wait_tm(1)
            ends_b1 = compact(1)
            fire_rows(1, ends_b1[-1])
            load_tm(ci + 2, 0)
            drain_rows(0, carry[-1])
            accum(0, carry, ci)

            wait_tm(0)
            ends_b0 = compact(0)
            fire_rows(0, ends_b0[-1])
            load_tm(ci + 3, 1)
            drain_rows(1, ends_b1[-1])
            accum(1, ends_b1, ci + 1)
            return ends_b0
        carry = lax.fori_loop(0, NCHUNK // 2 - 1, pair_body, ends0)

        wait_tm(1)
        ends_last = compact(1)
        fire_rows(1, ends_last[-1])
        drain_rows(0, carry[-1])
        accum(0, carry, NCHUNK - 2)
        drain_rows(1, ends_last[-1])
        accum(1, ends_last, NCHUNK - 1)

    return enc


def kernel(tokens, mask, W):
    B, S = tokens.shape
    V, D = W.shape
    enc = _build(B, S, D, V)
    out = enc(tokens.reshape(-1),
              mask.astype(jnp.int32).reshape(-1),
              W)
    return out.reshape(B, D)


# speedup vs baseline: 4.5235x; 1.0649x over previous
import functools

import jax
import jax.numpy as jnp
from jax import lax
from jax.experimental import pallas as pl
from jax.experimental.pallas import tpu as pltpu
from jax.experimental.pallas import tpu_sc as plsc

NW = 32
L = 16


@functools.lru_cache(maxsize=None)
def _build(B, S, D, V):
    SPW = B // NW
    C = 8
    NCHUNK = SPW // C
    CHW = C * S
    GB = 80
    NBLK = CHW // GB

    mesh = plsc.VectorSubcoreMesh(core_axis_name="c", subcore_axis_name="s")

    @functools.partial(
        pl.kernel,
        out_type=jax.ShapeDtypeStruct((B * D,), jnp.float32),
        mesh=mesh,
        scratch_types=[
            pltpu.VMEM((2 * CHW,), jnp.int32),
            pltpu.VMEM((2 * CHW,), jnp.int32),
            pltpu.VMEM((2 * CHW,), jnp.int32),
            pltpu.VMEM((2 * CHW, D), jnp.float32),
            pltpu.VMEM((C * D,), jnp.float32),
            pltpu.SemaphoreType.DMA,
            pltpu.SemaphoreType.DMA,
            pltpu.SemaphoreType.DMA,
            pltpu.SemaphoreType.DMA,
        ],
        compiler_params=pltpu.CompilerParams(
            use_tc_tiling_on_sc=False, needs_layout_passes=False),
    )
    def enc(tok_hbm, mask_hbm, table_hbm, out_hbm,
            tok_v, mask_v, idx_v, rows_v, outb_v,
            sem_r0, sem_r1, sem_t0, sem_t1):
        cid = lax.axis_index("c")
        sid = lax.axis_index("s")
        wid = sid * 2 + cid

        lanes = lax.iota(jnp.int32, L)
        first8 = jnp.minimum(jnp.maximum(8 - lanes, 0), 1)
        last8 = 1 - first8
        zero16 = jnp.zeros((L,), jnp.int32)
        zf16 = jnp.zeros((L,), jnp.float32)
        sem_r = (sem_r0, sem_r1)
        sem_t = (sem_t0, sem_t1)

        def seed_body(kk, c2):
            for u in range(4):
                off = (kk * 4 + u) * L
                idx_v[pl.ds(off, L)] = lanes + off
            return c2
        lax.fori_loop(0, 2 * CHW // (4 * L), seed_body, 0)

        def load_tm(ci, b):
            tbase = (wid * SPW + ci * C) * S
            tb = b * CHW
            pltpu.make_async_copy(
                tok_hbm.at[pl.ds(tbase, CHW)],
                tok_v.at[pl.ds(tb, CHW)], sem_t[b]).start()
            pltpu.make_async_copy(
                mask_hbm.at[pl.ds(tbase, CHW)],
                mask_v.at[pl.ds(tb, CHW)], sem_t[b]).start()

        def wait_tm(b):
            tb = b * CHW
            pltpu.make_async_copy(
                tok_hbm.at[pl.ds(0, CHW)],
                tok_v.at[pl.ds(tb, CHW)], sem_t[b]).wait()
            pltpu.make_async_copy(
                mask_hbm.at[pl.ds(0, CHW)],
                mask_v.at[pl.ds(tb, CHW)], sem_t[b]).wait()

        def compact(b):
            tb = b * CHW

            def emit(off_vec, k, sub):
                t = tok_v[pl.ds(tb + k * L, L)]
                m = mask_v[pl.ds(tb + k * L, L)]
                if sub == 0:
                    ms = m * first8
                elif sub == 1:
                    ms = m * last8
                else:
                    ms = m
                mb = ms != zero16
                pos = off_vec + plsc.cumsum(ms) - 1
                plsc.store_scatter(idx_v.at[pl.ds(tb, CHW)], [pos], t,
                                   mask=mb)
                return off_vec + plsc.all_reduce_population_count(mb)

            off_vec = zero16
            ends = []
            vpp = (2 * S) // L
            for p in range(C // 2):
                base = p * vpp
                for k in range(S // L):
                    off_vec = emit(off_vec, base + k, 2)
                off_vec = emit(off_vec, base + S // L, 0)
                ends.append(jnp.max(off_vec))
                off_vec = emit(off_vec, base + S // L, 1)
                for k in range(S // L + 1, vpp):
                    off_vec = emit(off_vec, base + k, 2)
                ends.append(jnp.max(off_vec))
            return tuple(ends)

        def fire_rows(b, ntot):
            tb = b * CHW

            def fire(bb, c2):
                @pl.when(bb * GB < ntot)
                def _():
                    pltpu.make_async_copy(
                        table_hbm.at[idx_v.at[pl.ds(tb + bb * GB, GB)]],
                        rows_v.at[pl.ds(tb + bb * GB, GB)], sem_r[b]).start()
                return c2
            lax.fori_loop(0, NBLK, fire, 0)

        def drain_rows(b, ntot):
            tb = b * CHW

            def drain(bb, c2):
                @pl.when(bb * GB < ntot)
                def _():
                    pltpu.make_async_copy(
                        table_hbm.at[idx_v.at[pl.ds(tb, GB)]],
                        rows_v.at[pl.ds(tb, GB)], sem_r[b]).wait()
                return c2
            lax.fori_loop(0, NBLK, drain, 0)

        def accum(b, ends, ci):
            rbb = b * CHW
            for s in range(C):
                off_s = jnp.int32(0) if s == 0 else ends[s - 1]
                cnt_s = ends[s] - off_s
                rb0 = rbb + off_s

                def g8(g, a, rb0=rb0):
                    a0, a1 = a
                    rb = rb0 + g * 8
                    for r in range(8):
                        a0 = a0 + rows_v[rb + r, pl.ds(0, L)]
                        a1 = a1 + rows_v[rb + r, pl.ds(L, L)]
                    return (a0, a1)
                acc0, acc1 = lax.fori_loop(0, cnt_s // 8, g8, (zf16, zf16))

                def g1(j, a, rb0=rb0):
                    a0, a1 = a
                    return (a0 + rows_v[rb0 + j, pl.ds(0, L)],
                            a1 + rows_v[rb0 + j, pl.ds(L, L)])
                acc0, acc1 = lax.fori_loop(
                    (cnt_s // 8) * 8, cnt_s, g1, (acc0, acc1))

                cntf = jnp.full((L,), cnt_s.astype(jnp.float32))
                scale = jnp.float32(1.0) / jnp.maximum(cntf, 1.0)
                outb_v[pl.ds(s * D, L)] = acc0 * scale
                outb_v[pl.ds(s * D + L, L)] = acc1 * scale
            pltpu.sync_copy(
                outb_v, out_hbm.at[pl.ds((wid * SPW + ci * C) * D, C * D)])

        load_tm(0, 0)
        wait_tm(0)
        ends0 = compact(0)
        fire_rows(0, ends0[-1])
        load_tm(1, 1)

        def pair_body(pi, carry):
            ci = 2 * pi
            wait_tm(1)
            ends_b1 = compact(1)
            fire_rows(1, ends_b1[-1])
            load_tm(ci + 2, 0)
            drain_rows(0, carry[-1])
            accum(0, carry, ci)

            wait_tm(0)
            ends_b0 = compact(0)
            fire_rows(0, ends_b0[-1])
            load_tm(ci + 3, 1)
            drain_rows(1, ends_b1[-1])
            accum(1, ends_b1, ci + 1)
            return ends_b0
        carry = lax.fori_loop(0, NCHUNK // 2 - 1, pair_body, ends0)

        wait_tm(1)
        ends_last = compact(1)
        fire_rows(1, ends_last[-1])
        drain_rows(0, carry[-1])
        accum(0, carry, NCHUNK - 2)
        drain_rows(1, ends_last[-1])
        accum(1, ends_last, NCHUNK - 1)

    return enc


def kernel(tokens, mask, W):
    B, S = tokens.shape
    V, D = W.shape
    enc = _build(B, S, D, V)
    out = enc(tokens.reshape(-1),
              mask.astype(jnp.int32).reshape(-1),
              W)
    return out.reshape(B, D)
